# baseline (speedup 1.0000x reference)
import functools

import jax
import jax.numpy as jnp
from jax import lax
from jax.experimental import pallas as pl
from jax.experimental.pallas import tpu as pltpu
from jax.experimental.pallas import tpu_sc as plsc

N = 4096
C = 1000
RB = 256
NBLK = N // RB
NEG_INF = float("-inf")
POS_INF = float("inf")


def _stats_body(x_ref, lab_ref, conf_ref, hits_ref):
    x = x_ref[...]
    lab = lab_ref[0, :]
    maxv = jnp.max(x, axis=1, keepdims=True)
    s = jnp.sum(jnp.exp(x - maxv), axis=1)
    col = lax.broadcasted_iota(jnp.int32, x.shape, 1)
    am = jnp.min(jnp.where(x >= maxv, col, C), axis=1)
    conf_ref[0, :] = 1.0 / s
    hits_ref[0, :] = (am == lab).astype(jnp.float32)


SB = 512


def _stats(x, lab2):
    return pl.pallas_call(
        _stats_body,
        grid=(N // SB,),
        in_specs=[
            pl.BlockSpec((SB, C), lambda i: (i, 0)),
            pl.BlockSpec((1, SB), lambda i: (0, i)),
        ],
        out_specs=[
            pl.BlockSpec((1, SB), lambda i: (0, i)),
            pl.BlockSpec((1, SB), lambda i: (0, i)),
        ],
        out_shape=[
            jax.ShapeDtypeStruct((1, N), jnp.float32),
            jax.ShapeDtypeStruct((1, N), jnp.float32),
        ],
    )(x, lab2)


def _rank_body(conf_ref, rank_ref):
    i = pl.program_id(0)
    cj = conf_ref[0, :].reshape(1, N)
    ci = conf_ref[0, pl.ds(i * RB, RB)].reshape(RB, 1)
    jidx = lax.broadcasted_iota(jnp.int32, (RB, N), 1)
    iidx = i * RB + lax.broadcasted_iota(jnp.int32, (RB, N), 0)
    before = (cj < ci) | ((cj == ci) & (jidx < iidx))
    rank_ref[0, :] = jnp.sum(before.astype(jnp.int32), axis=1)


def _ranks(conf):
    return pl.pallas_call(
        _rank_body,
        grid=(NBLK,),
        in_specs=[pl.BlockSpec((1, N), lambda i: (0, 0))],
        out_specs=pl.BlockSpec((1, RB), lambda i: (0, i)),
        out_shape=jax.ShapeDtypeStruct((1, N), jnp.int32),
    )(conf)


def _sc_scatter_scan(rank, hits):
    mesh = plsc.VectorSubcoreMesh(core_axis_name="c", subcore_axis_name="s")

    @functools.partial(
        pl.kernel,
        mesh=mesh,
        compiler_params=pltpu.CompilerParams(needs_layout_passes=False),
        out_type=[
            jax.ShapeDtypeStruct((N,), jnp.float32),
            jax.ShapeDtypeStruct((N,), jnp.float32),
        ],
        scratch_types=[
            pltpu.VMEM((N,), jnp.int32),
            pltpu.VMEM((N,), jnp.float32),
            pltpu.VMEM((N,), jnp.float32),
            pltpu.VMEM((N,), jnp.float32),
        ],
    )
    def sc_kernel(rank_hbm, hits_hbm, hs_out, cinc_out, rank_v, hits_v,
                  hs_v, cs_v):
        cid = lax.axis_index("c")
        sid = lax.axis_index("s")

        @pl.when(jnp.logical_and(cid == 0, sid == 0))
        def _():
            pltpu.sync_copy(rank_hbm, rank_v)
            pltpu.sync_copy(hits_hbm, hits_v)

            def scatter_body(i, carry):
                kv = rank_v[pl.ds(i * 16, 16)]
                hv = hits_v[pl.ds(i * 16, 16)]
                plsc.store_scatter(hs_v, [kv], hv)
                return carry

            lax.fori_loop(0, N // 16, scatter_body, 0)

            def scan_body(i, carry):
                hv = hs_v[pl.ds(i * 16, 16)]
                cs = plsc.cumsum(hv) + carry
                cs_v[pl.ds(i * 16, 16)] = cs
                return carry + jnp.sum(hv)

            lax.fori_loop(0, N // 16, scan_body, jnp.float32(0.0))

            pltpu.sync_copy(hs_v, hs_out)
            pltpu.sync_copy(cs_v, cinc_out)

    return sc_kernel(rank, hits)


KT = 256


def _pava_body(cinc_ref, hs_ref, out_ref, carry_ref, acc_ref):
    b = pl.program_id(0)
    base = b * RB

    @pl.when(b == 0)
    def _():
        carry_ref[...] = jnp.full((1, N), NEG_INF, jnp.float32)

    sexc = (cinc_ref[0, pl.ds(base, RB)]
            - hs_ref[0, pl.ds(base, RB)]).reshape(RB, 1)
    rowiota = lax.broadcasted_iota(jnp.int32, (RB, KT), 0)
    coliota = lax.broadcasted_iota(jnp.int32, (RB, KT), 1)
    acc_ref[...] = jnp.full((RB, 1), POS_INF, jnp.float32)

    def tile_body(t, c):
        k0 = t * KT
        cinc_t = cinc_ref[0, pl.ds(k0, KT)].reshape(1, KT)
        kk = k0 + coliota
        jj = base + rowiota
        length = kk - jj + 1
        valid = length >= 1
        denom = jnp.where(valid, length, 1).astype(jnp.float32)
        M = jnp.where(valid, (cinc_t - sexc) / denom, NEG_INF)

        T = M
        s = 1
        while s < RB:
            r1 = jnp.where(rowiota >= s,
                           pltpu.roll(T, s, axis=0), NEG_INF)
            r2 = jnp.where(rowiota >= 2 * s,
                           pltpu.roll(T, 2 * s, axis=0), NEG_INF)
            r3 = jnp.where(rowiota >= 3 * s,
                           pltpu.roll(T, 3 * s, axis=0), NEG_INF)
            T = jnp.maximum(jnp.maximum(T, r1), jnp.maximum(r2, r3))
            s *= 4
        T = jnp.maximum(T, carry_ref[0, pl.ds(k0, KT)].reshape(1, KT))
        carry_ref[0, pl.ds(k0, KT)] = T[RB - 1, :]

        m = jnp.where(kk >= jj, T, POS_INF)
        acc_ref[...] = jnp.minimum(acc_ref[...],
                                   jnp.min(m, axis=1, keepdims=True))
        return c

    lax.fori_loop(b * RB // KT, N // KT, tile_body, 0)
    out_ref[...] = acc_ref[...]


def _pava(cinc, hs):
    return pl.pallas_call(
        _pava_body,
        grid=(NBLK,),
        in_specs=[
            pl.BlockSpec((1, N), lambda i: (0, 0)),
            pl.BlockSpec((1, N), lambda i: (0, 0)),
        ],
        out_specs=pl.BlockSpec((RB, 1), lambda i: (i, 0)),
        out_shape=jax.ShapeDtypeStruct((N, 1), jnp.float32),
        scratch_shapes=[pltpu.VMEM((1, N), jnp.float32),
                        pltpu.VMEM((RB, 1), jnp.float32)],
    )(cinc, hs)


def kernel(Simple_vector, label_list):
    lab2 = label_list.reshape(1, N)
    conf, hits = _stats(Simple_vector, lab2)
    rank = _ranks(conf)
    hits_s, cinc = _sc_scatter_scan(rank.reshape(N), hits.reshape(N))
    cali = _pava(cinc.reshape(1, N), hits_s.reshape(1, N))
    return cali.reshape(N), hits_s > jnp.float32(0.5)


__all__ = ["kernel"]


# speedup vs baseline: 1.1153x; 1.1153x over previous
import functools

import jax
import jax.numpy as jnp
from jax import lax
from jax.experimental import pallas as pl
from jax.experimental.pallas import tpu as pltpu
from jax.experimental.pallas import tpu_sc as plsc

N = 4096
C = 1000
RB = 256
NBLK = N // RB
NEG_INF = float("-inf")
POS_INF = float("inf")


def _stats_body(x_ref, lab_ref, conf_ref, hits_ref):
    x = x_ref[...]
    lab = lab_ref[0, :]
    maxv = jnp.max(x, axis=1, keepdims=True)
    s = jnp.sum(jnp.exp(x - maxv), axis=1)
    col = lax.broadcasted_iota(jnp.int32, x.shape, 1)
    am = jnp.min(jnp.where(x >= maxv, col, C), axis=1)
    conf_ref[0, :] = 1.0 / s
    hits_ref[0, :] = (am == lab).astype(jnp.float32)


SB = 512


def _stats(x, lab2):
    return pl.pallas_call(
        _stats_body,
        grid=(N // SB,),
        in_specs=[
            pl.BlockSpec((SB, C), lambda i: (i, 0)),
            pl.BlockSpec((1, SB), lambda i: (0, i)),
        ],
        out_specs=[
            pl.BlockSpec((1, SB), lambda i: (0, i)),
            pl.BlockSpec((1, SB), lambda i: (0, i)),
        ],
        out_shape=[
            jax.ShapeDtypeStruct((1, N), jnp.float32),
            jax.ShapeDtypeStruct((1, N), jnp.float32),
        ],
    )(x, lab2)


def _rank_body(conf_ref, rank_ref):
    i = pl.program_id(0)
    cj = conf_ref[0, :].reshape(1, N)
    ci = conf_ref[0, pl.ds(i * RB, RB)].reshape(RB, 1)
    jidx = lax.broadcasted_iota(jnp.int32, (RB, N), 1)
    iidx = i * RB + lax.broadcasted_iota(jnp.int32, (RB, N), 0)
    before = (cj < ci) | ((cj == ci) & (jidx < iidx))
    rank_ref[0, :] = jnp.sum(before.astype(jnp.int32), axis=1)


def _ranks(conf):
    return pl.pallas_call(
        _rank_body,
        grid=(NBLK,),
        in_specs=[pl.BlockSpec((1, N), lambda i: (0, 0))],
        out_specs=pl.BlockSpec((1, RB), lambda i: (0, i)),
        out_shape=jax.ShapeDtypeStruct((1, N), jnp.int32),
    )(conf)


def _sc_scatter_scan(rank, hits):
    mesh = plsc.VectorSubcoreMesh(core_axis_name="c", subcore_axis_name="s")

    @functools.partial(
        pl.kernel,
        mesh=mesh,
        compiler_params=pltpu.CompilerParams(needs_layout_passes=False),
        out_type=[
            jax.ShapeDtypeStruct((N,), jnp.float32),
            jax.ShapeDtypeStruct((N,), jnp.float32),
        ],
        scratch_types=[
            pltpu.VMEM((N,), jnp.int32),
            pltpu.VMEM((N,), jnp.float32),
            pltpu.VMEM((N,), jnp.float32),
            pltpu.VMEM((N,), jnp.float32),
        ],
    )
    def sc_kernel(rank_hbm, hits_hbm, hs_out, cinc_out, rank_v, hits_v,
                  hs_v, cs_v):
        cid = lax.axis_index("c")
        sid = lax.axis_index("s")

        @pl.when(jnp.logical_and(cid == 0, sid == 0))
        def _():
            pltpu.sync_copy(rank_hbm, rank_v)
            pltpu.sync_copy(hits_hbm, hits_v)

            def scatter_body(i, carry):
                kv = rank_v[pl.ds(i * 16, 16)]
                hv = hits_v[pl.ds(i * 16, 16)]
                plsc.store_scatter(hs_v, [kv], hv)
                return carry

            lax.fori_loop(0, N // 16, scatter_body, 0)

            def scan_body(i, carry):
                hv = hs_v[pl.ds(i * 16, 16)]
                cs = plsc.cumsum(hv) + carry
                cs_v[pl.ds(i * 16, 16)] = cs
                return carry + jnp.sum(hv)

            lax.fori_loop(0, N // 16, scan_body, jnp.float32(0.0))

            pltpu.sync_copy(hs_v, hs_out)
            pltpu.sync_copy(cs_v, cinc_out)

    return sc_kernel(rank, hits)


KT = 256


def _cummax_rows(T):
    s = 1
    while s < RB:
        T = jnp.maximum(T, jnp.concatenate(
            [jnp.full((s, KT), NEG_INF, jnp.float32), T[: RB - s, :]],
            axis=0))
        s *= 2
    return T


def _pava_body(cinc_ref, hs_ref, out_ref, hsb_ref, carry_ref, acc_ref):
    b = pl.program_id(0)
    base = b * RB

    @pl.when(b == 0)
    def _():
        carry_ref[...] = jnp.full((1, N), NEG_INF, jnp.float32)
        hsb_ref[...] = hs_ref[...] > 0.5

    sexc = (cinc_ref[0, pl.ds(base, RB)]
            - hs_ref[0, pl.ds(base, RB)]).reshape(RB, 1)
    dif = (lax.broadcasted_iota(jnp.int32, (RB, KT), 1)
           - lax.broadcasted_iota(jnp.int32, (RB, KT), 0)
           ).astype(jnp.float32)

    cinc_t = cinc_ref[0, pl.ds(base, KT)].reshape(1, KT)
    valid = dif >= 0.0
    M = jnp.where(valid, (cinc_t - sexc) / jnp.where(valid, dif + 1.0, 1.0),
                  NEG_INF)
    T = jnp.maximum(_cummax_rows(M), carry_ref[0, pl.ds(base, KT)]
                    .reshape(1, KT))
    carry_ref[0, pl.ds(base, KT)] = T[RB - 1, :]
    acc_ref[...] = jnp.min(jnp.where(valid, T, POS_INF), axis=1,
                           keepdims=True)

    def tile_body(t, c):
        k0 = t * KT
        cinc_t = cinc_ref[0, pl.ds(k0, KT)].reshape(1, KT)
        M = (cinc_t - sexc) / (dif + (k0 - base + 1.0))
        T = jnp.maximum(_cummax_rows(M), carry_ref[0, pl.ds(k0, KT)]
                        .reshape(1, KT))
        carry_ref[0, pl.ds(k0, KT)] = T[RB - 1, :]
        acc_ref[...] = jnp.minimum(acc_ref[...],
                                   jnp.min(T, axis=1, keepdims=True))
        return c

    lax.fori_loop(b + 1, N // KT, tile_body, 0)
    out_ref[...] = acc_ref[...]


def _pava(cinc, hs):
    return pl.pallas_call(
        _pava_body,
        grid=(NBLK,),
        in_specs=[
            pl.BlockSpec((1, N), lambda i: (0, 0)),
            pl.BlockSpec((1, N), lambda i: (0, 0)),
        ],
        out_specs=[pl.BlockSpec((RB, 1), lambda i: (i, 0)),
                   pl.BlockSpec((1, N), lambda i: (0, 0))],
        out_shape=[jax.ShapeDtypeStruct((N, 1), jnp.float32),
                   jax.ShapeDtypeStruct((1, N), jnp.bool_)],
        scratch_shapes=[pltpu.VMEM((1, N), jnp.float32),
                        pltpu.VMEM((RB, 1), jnp.float32)],
    )(cinc, hs)


def kernel(Simple_vector, label_list):
    lab2 = label_list.reshape(1, N)
    conf, hits = _stats(Simple_vector, lab2)
    rank = _ranks(conf)
    hits_s, cinc = _sc_scatter_scan(rank.reshape(N), hits.reshape(N))
    cali, hitsb = _pava(cinc.reshape(1, N), hits_s.reshape(1, N))
    return cali.reshape(N), hitsb.reshape(N)


__all__ = ["kernel"]


# speedup vs baseline: 1.7817x; 1.5975x over previous
import functools

import jax
import jax.numpy as jnp
from jax import lax
from jax.experimental import pallas as pl
from jax.experimental.pallas import tpu as pltpu
from jax.experimental.pallas import tpu_sc as plsc

N = 4096
C = 1000
RB = 256
NBLK = N // RB
NEG_INF = float("-inf")
POS_INF = float("inf")


def _stats_body(x_ref, lab_ref, conf_ref, hits_ref):
    x = x_ref[...]
    lab = lab_ref[0, :]
    maxv = jnp.max(x, axis=1, keepdims=True)
    s = jnp.sum(jnp.exp(x - maxv), axis=1)
    col = lax.broadcasted_iota(jnp.int32, x.shape, 1)
    am = jnp.min(jnp.where(x >= maxv, col, C), axis=1)
    conf_ref[0, :] = 1.0 / s
    hits_ref[0, :] = (am == lab).astype(jnp.float32)


SB = 512


def _stats(x, lab2):
    return pl.pallas_call(
        _stats_body,
        grid=(N // SB,),
        in_specs=[
            pl.BlockSpec((SB, C), lambda i: (i, 0)),
            pl.BlockSpec((1, SB), lambda i: (0, i)),
        ],
        out_specs=[
            pl.BlockSpec((1, SB), lambda i: (0, i)),
            pl.BlockSpec((1, SB), lambda i: (0, i)),
        ],
        out_shape=[
            jax.ShapeDtypeStruct((1, N), jnp.float32),
            jax.ShapeDtypeStruct((1, N), jnp.float32),
        ],
    )(x, lab2)


def _rank_body(conf_ref, rank_ref):
    i = pl.program_id(0)
    cj = conf_ref[0, :].reshape(1, N)
    ci = conf_ref[0, pl.ds(i * RB, RB)].reshape(RB, 1)
    jidx = lax.broadcasted_iota(jnp.int32, (RB, N), 1)
    iidx = i * RB + lax.broadcasted_iota(jnp.int32, (RB, N), 0)
    before = (cj < ci) | ((cj == ci) & (jidx < iidx))
    rank_ref[0, :] = jnp.sum(before.astype(jnp.int32), axis=1)


def _ranks(conf):
    return pl.pallas_call(
        _rank_body,
        grid=(NBLK,),
        in_specs=[pl.BlockSpec((1, N), lambda i: (0, 0))],
        out_specs=pl.BlockSpec((1, RB), lambda i: (0, i)),
        out_shape=jax.ShapeDtypeStruct((1, N), jnp.int32),
    )(conf)


def _sc_scatter_scan(rank, hits):
    mesh = plsc.VectorSubcoreMesh(core_axis_name="c", subcore_axis_name="s")

    @functools.partial(
        pl.kernel,
        mesh=mesh,
        compiler_params=pltpu.CompilerParams(needs_layout_passes=False),
        out_type=[
            jax.ShapeDtypeStruct((N,), jnp.float32),
            jax.ShapeDtypeStruct((N,), jnp.float32),
        ],
        scratch_types=[
            pltpu.VMEM((N,), jnp.int32),
            pltpu.VMEM((N,), jnp.float32),
            pltpu.VMEM((N,), jnp.float32),
            pltpu.VMEM((N,), jnp.float32),
        ],
    )
    def sc_kernel(rank_hbm, hits_hbm, hs_out, cinc_out, rank_v, hits_v,
                  hs_v, cs_v):
        cid = lax.axis_index("c")
        sid = lax.axis_index("s")

        @pl.when(jnp.logical_and(cid == 0, sid == 0))
        def _():
            pltpu.sync_copy(rank_hbm, rank_v)
            pltpu.sync_copy(hits_hbm, hits_v)

            def scatter_body(i, carry):
                kv = rank_v[pl.ds(i * 16, 16)]
                hv = hits_v[pl.ds(i * 16, 16)]
                plsc.store_scatter(hs_v, [kv], hv)
                return carry

            lax.fori_loop(0, N // 16, scatter_body, 0)

            def scan_body(i, carry):
                hv = hs_v[pl.ds(i * 16, 16)]
                cs = plsc.cumsum(hv) + carry
                cs_v[pl.ds(i * 16, 16)] = cs
                return carry + jnp.sum(hv)

            lax.fori_loop(0, N // 16, scan_body, jnp.float32(0.0))

            pltpu.sync_copy(hs_v, hs_out)
            pltpu.sync_copy(cs_v, cinc_out)

    return sc_kernel(rank, hits)


KT = 256


def _cummax_rows(T):
    s = 1
    while s < RB:
        T = jnp.maximum(T, jnp.concatenate(
            [jnp.full((s, KT), NEG_INF, jnp.float32), T[: RB - s, :]],
            axis=0))
        s *= 2
    return T


def _pava_body(cinc_ref, hs_ref, out_ref, hsb_ref, carry_ref, acc_ref):
    b = pl.program_id(0)
    base = b * RB

    @pl.when(b == 0)
    def _():
        carry_ref[...] = jnp.full((1, N), NEG_INF, jnp.float32)
        hsb_ref[...] = hs_ref[...] > 0.5

    sexc = (cinc_ref[0, pl.ds(base, RB)]
            - hs_ref[0, pl.ds(base, RB)]).reshape(RB, 1)
    dif = (lax.broadcasted_iota(jnp.int32, (RB, KT), 1)
           - lax.broadcasted_iota(jnp.int32, (RB, KT), 0)
           ).astype(jnp.float32)

    cinc_t = cinc_ref[0, pl.ds(base, KT)].reshape(1, KT)
    valid = dif >= 0.0
    M = jnp.where(valid, (cinc_t - sexc) / jnp.where(valid, dif + 1.0, 1.0),
                  NEG_INF)
    T = jnp.maximum(_cummax_rows(M), carry_ref[0, pl.ds(base, KT)]
                    .reshape(1, KT))
    carry_ref[0, pl.ds(base, KT)] = T[RB - 1, :]
    acc_ref[...] = jnp.min(jnp.where(valid, T, POS_INF), axis=1,
                           keepdims=True)

    def tile_body(t, c):
        k0 = t * KT
        cinc_t = cinc_ref[0, pl.ds(k0, KT)].reshape(1, KT)
        M = (cinc_t - sexc) / (dif + (k0 - base + 1.0))
        T = jnp.maximum(_cummax_rows(M), carry_ref[0, pl.ds(k0, KT)]
                        .reshape(1, KT))
        carry_ref[0, pl.ds(k0, KT)] = T[RB - 1, :]
        acc_ref[...] = jnp.minimum(acc_ref[...],
                                   jnp.min(T, axis=1, keepdims=True))
        return c

    lax.fori_loop(b + 1, N // KT, tile_body, 0)
    out_ref[...] = acc_ref[...]


def _pava(cinc, hs):
    return pl.pallas_call(
        _pava_body,
        grid=(NBLK,),
        in_specs=[
            pl.BlockSpec((1, N), lambda i: (0, 0)),
            pl.BlockSpec((1, N), lambda i: (0, 0)),
        ],
        out_specs=[pl.BlockSpec((RB, 1), lambda i: (i, 0)),
                   pl.BlockSpec((1, N), lambda i: (0, 0))],
        out_shape=[jax.ShapeDtypeStruct((N, 1), jnp.float32),
                   jax.ShapeDtypeStruct((1, N), jnp.bool_)],
        scratch_shapes=[pltpu.VMEM((1, N), jnp.float32),
                        pltpu.VMEM((RB, 1), jnp.float32)],
    )(cinc, hs)


HMAX = 512


def _sc_compact(conf, hits):
    mesh = plsc.VectorSubcoreMesh(core_axis_name="c", subcore_axis_name="s")

    @functools.partial(
        pl.kernel,
        mesh=mesh,
        compiler_params=pltpu.CompilerParams(needs_layout_passes=False),
        out_type=[
            jax.ShapeDtypeStruct((HMAX,), jnp.float32),
            jax.ShapeDtypeStruct((HMAX,), jnp.int32),
            jax.ShapeDtypeStruct((16,), jnp.float32),
        ],
        scratch_types=[
            pltpu.VMEM((N,), jnp.float32),
            pltpu.VMEM((N,), jnp.float32),
            pltpu.VMEM((HMAX,), jnp.float32),
            pltpu.VMEM((HMAX,), jnp.int32),
            pltpu.VMEM((16,), jnp.float32),
        ],
    )
    def sc_kernel(conf_hbm, hits_hbm, ch_out, ih_out, aux_out,
                  conf_v, hits_v, ch_v, ih_v, aux_v):
        cid = lax.axis_index("c")
        sid = lax.axis_index("s")

        @pl.when(jnp.logical_and(cid == 0, sid == 0))
        def _():
            pltpu.sync_copy(conf_hbm, conf_v)
            pltpu.sync_copy(hits_hbm, hits_v)

            def pad_body(i, c):
                ch_v[pl.ds(i * 16, 16)] = jnp.full((16,), 2.0, jnp.float32)
                ih_v[pl.ds(i * 16, 16)] = jnp.zeros((16,), jnp.int32)
                return c

            lax.fori_loop(0, HMAX // 16, pad_body, 0)

            lane = lax.iota(jnp.int32, 16)

            def comp_body(i, off):
                cv = conf_v[pl.ds(i * 16, 16)]
                hv = hits_v[pl.ds(i * 16, 16)]
                mask = hv > 0.5
                pos = off + plsc.cumsum(jnp.where(mask, 1, 0)) - 1
                plsc.store_scatter(ch_v, [pos], cv, mask=mask)
                plsc.store_scatter(ih_v, [pos], i * 16 + lane, mask=mask)
                npop = plsc.all_reduce_population_count(mask)
                return off + jnp.max(npop)

            cnt = lax.fori_loop(0, N // 16, comp_body, jnp.int32(0))
            aux_v[...] = jnp.where(lane == 0, cnt.astype(jnp.float32), 0.0)

            pltpu.sync_copy(ch_v, ch_out)
            pltpu.sync_copy(ih_v, ih_out)
            pltpu.sync_copy(aux_v, aux_out)

    return sc_kernel(conf, hits)


def _solve_body(chc_ref, ihc_ref, chr_ref, ihr_ref, aux_ref, conf_ref,
                out_ref, hsb_ref):
    cnt_i = aux_ref[0, 0].astype(jnp.int32)

    ch_c = chc_ref[...]
    ih_c = ihc_ref[...]
    ch_r = chr_ref[...]
    ih_r = ihr_ref[...]

    CHK = 1024
    jcol = lax.broadcasted_iota(jnp.int32, (HMAX, CHK), 1)

    def rank_chunk(t, racc):
        cj = conf_ref[0, pl.ds(t * CHK, CHK)].reshape(1, CHK)
        jj = t * CHK + jcol
        before = (cj < ch_c) | ((cj == ch_c) & (jj < ih_c))
        return racc + jnp.sum(before.astype(jnp.int32), axis=1, keepdims=True)

    rh_c = lax.fori_loop(0, N // CHK, rank_chunk,
                         jnp.zeros((HMAX, 1), jnp.int32))

    CHR = 512
    jrow2 = lax.broadcasted_iota(jnp.int32, (CHR, 1), 0)

    def rank_chunk_r(t, racc):
        cj = conf_ref[0, pl.ds(t * CHR, CHR)].reshape(CHR, 1)
        jj = t * CHR + jrow2
        before = (cj < ch_r) | ((cj == ch_r) & (jj < ih_r))
        return racc + jnp.sum(before.astype(jnp.int32), axis=0, keepdims=True)

    rh_r = lax.fori_loop(0, N // CHR, rank_chunk_r,
                         jnp.zeros((1, HMAX), jnp.int32))

    hcol = lax.broadcasted_iota(jnp.int32, (HMAX, 1), 0)
    hrow = lax.broadcasted_iota(jnp.int32, (1, HMAX), 1)
    before2 = (rh_r < rh_c) | ((rh_r == rh_c) & (hrow < hcol))
    rr_c = jnp.sum(before2.astype(jnp.int32), axis=1, keepdims=True)
    before2t = (rh_c < rh_r) | ((rh_c == rh_r) & (hcol < hrow))
    rr_r = jnp.sum(before2t.astype(jnp.int32), axis=0, keepdims=True)

    m_hits_r = jnp.sum(rh_c.astype(jnp.float32)
                       * (rr_c == hrow).astype(jnp.float32),
                       axis=0, keepdims=True)
    m_hits_c = jnp.sum(rh_r.astype(jnp.float32)
                       * (rr_r == hcol).astype(jnp.float32),
                       axis=1, keepdims=True)

    candm_r = jnp.concatenate(
        [jnp.zeros((1, 1), jnp.float32), m_hits_r[:, : HMAX - 1]], axis=1)
    candm_c = jnp.concatenate(
        [jnp.zeros((1, 1), jnp.float32), m_hits_c[: HMAX - 1, :]], axis=0)
    candS_r = jnp.maximum(hrow - 1, 0).astype(jnp.float32)
    candS_c = jnp.maximum(hcol - 1, 0).astype(jnp.float32)

    fh = jnp.sum(jnp.where(hrow == 0, m_hits_r, 0.0))
    r0_ok = fh != 0.0
    vr_c = (hcol <= cnt_i) & ((hcol >= 1) | r0_ok)
    validq = (hrow >= 1) & (hrow <= cnt_i + 1)

    qgtr = hrow > hcol
    M = jnp.where(qgtr & validq & vr_c,
                  (candS_r - candS_c) / (candm_r - candm_c), NEG_INF)

    s = 1
    while s < HMAX:
        M = jnp.maximum(M, jnp.concatenate(
            [jnp.full((s, HMAX), NEG_INF, jnp.float32), M[: HMAX - s, :]],
            axis=0))
        s *= 2
    slope_c = jnp.min(jnp.where(qgtr & validq, M, POS_INF), axis=1,
                      keepdims=True)

    lanes = lax.broadcasted_iota(jnp.int32, (1, RB), 1)
    cntmask_c = hcol <= cnt_i + 1

    def map_body(b2, c):
        i_row = (b2 * RB + lanes).astype(jnp.float32)
        pi = jnp.sum(((candm_c <= i_row) & cntmask_c).astype(jnp.int32),
                     axis=0, keepdims=True) - 1
        iso = jnp.sum(jnp.where(hcol == pi, slope_c, 0.0), axis=0,
                      keepdims=True)
        out_ref[0, pl.ds(b2 * RB, RB)] = iso[0, :]
        kk = b2 * RB + lanes
        hb = jnp.sum((rh_c == kk).astype(jnp.int32), axis=0,
                     keepdims=True) > 0
        hsb_ref[0, pl.ds(b2 * RB, RB)] = hb[0, :]
        return c

    lax.fori_loop(0, N // RB, map_body, 0)


def _solve(ch, ih, aux, conf):
    return pl.pallas_call(
        _solve_body,
        grid=(1,),
        in_specs=[
            pl.BlockSpec((HMAX, 1), lambda i: (0, 0)),
            pl.BlockSpec((HMAX, 1), lambda i: (0, 0)),
            pl.BlockSpec((1, HMAX), lambda i: (0, 0)),
            pl.BlockSpec((1, HMAX), lambda i: (0, 0)),
            pl.BlockSpec(memory_space=pltpu.SMEM),
            pl.BlockSpec((1, N), lambda i: (0, 0)),
        ],
        out_specs=[pl.BlockSpec((1, N), lambda i: (0, 0)),
                   pl.BlockSpec((1, N), lambda i: (0, 0))],
        out_shape=[jax.ShapeDtypeStruct((1, N), jnp.float32),
                   jax.ShapeDtypeStruct((1, N), jnp.bool_)],
    )(ch.reshape(HMAX, 1), ih.reshape(HMAX, 1),
      ch.reshape(1, HMAX), ih.reshape(1, HMAX), aux, conf)


def kernel(Simple_vector, label_list):
    lab2 = label_list.reshape(1, N)
    conf, hits = _stats(Simple_vector, lab2)

    def sparse_path(_):
        ch, ih, aux = _sc_compact(conf.reshape(N), hits.reshape(N))
        cali, hb = _solve(ch, ih, aux.reshape(1, 16), conf)
        return cali.reshape(N), hb.reshape(N)

    def dense_path(_):
        rank = _ranks(conf)
        hits_s, cinc = _sc_scatter_scan(rank.reshape(N), hits.reshape(N))
        cali, hb = _pava(cinc.reshape(1, N), hits_s.reshape(1, N))
        return cali.reshape(N), hb.reshape(N)

    nhits = jnp.sum(hits)
    return lax.cond(nhits <= HMAX - 2.0, sparse_path, dense_path, 0)


__all__ = ["kernel"]


# speedup vs baseline: 2.0186x; 1.1330x over previous
import functools

import jax
import jax.numpy as jnp
from jax import lax
from jax.experimental import pallas as pl
from jax.experimental.pallas import tpu as pltpu
from jax.experimental.pallas import tpu_sc as plsc

N = 4096
C = 1000
RB = 256
NBLK = N // RB
NEG_INF = float("-inf")
POS_INF = float("inf")


def _stats_body(x_ref, lab_ref, conf_ref, hits_ref):
    x = x_ref[...]
    lab = lab_ref[0, :]
    maxv = jnp.max(x, axis=1, keepdims=True)
    s = jnp.sum(jnp.exp(x - maxv), axis=1)
    col = lax.broadcasted_iota(jnp.int32, x.shape, 1)
    am = jnp.min(jnp.where(x >= maxv, col, C), axis=1)
    conf_ref[0, :] = 1.0 / s
    hits_ref[0, :] = (am == lab).astype(jnp.float32)


SB = 1024


def _stats(x, lab2):
    return pl.pallas_call(
        _stats_body,
        grid=(N // SB,),
        in_specs=[
            pl.BlockSpec((SB, C), lambda i: (i, 0)),
            pl.BlockSpec((1, SB), lambda i: (0, i)),
        ],
        out_specs=[
            pl.BlockSpec((1, SB), lambda i: (0, i)),
            pl.BlockSpec((1, SB), lambda i: (0, i)),
        ],
        out_shape=[
            jax.ShapeDtypeStruct((1, N), jnp.float32),
            jax.ShapeDtypeStruct((1, N), jnp.float32),
        ],
    )(x, lab2)


def _rank_body(conf_ref, rank_ref):
    i = pl.program_id(0)
    cj = conf_ref[0, :].reshape(1, N)
    ci = conf_ref[0, pl.ds(i * RB, RB)].reshape(RB, 1)
    jidx = lax.broadcasted_iota(jnp.int32, (RB, N), 1)
    iidx = i * RB + lax.broadcasted_iota(jnp.int32, (RB, N), 0)
    before = (cj < ci) | ((cj == ci) & (jidx < iidx))
    rank_ref[0, :] = jnp.sum(before.astype(jnp.int32), axis=1)


def _ranks(conf):
    return pl.pallas_call(
        _rank_body,
        grid=(NBLK,),
        in_specs=[pl.BlockSpec((1, N), lambda i: (0, 0))],
        out_specs=pl.BlockSpec((1, RB), lambda i: (0, i)),
        out_shape=jax.ShapeDtypeStruct((1, N), jnp.int32),
    )(conf)


def _sc_scatter_scan(rank, hits):
    mesh = plsc.VectorSubcoreMesh(core_axis_name="c", subcore_axis_name="s")

    @functools.partial(
        pl.kernel,
        mesh=mesh,
        compiler_params=pltpu.CompilerParams(needs_layout_passes=False),
        out_type=[
            jax.ShapeDtypeStruct((N,), jnp.float32),
            jax.ShapeDtypeStruct((N,), jnp.float32),
        ],
        scratch_types=[
            pltpu.VMEM((N,), jnp.int32),
            pltpu.VMEM((N,), jnp.float32),
            pltpu.VMEM((N,), jnp.float32),
            pltpu.VMEM((N,), jnp.float32),
        ],
    )
    def sc_kernel(rank_hbm, hits_hbm, hs_out, cinc_out, rank_v, hits_v,
                  hs_v, cs_v):
        cid = lax.axis_index("c")
        sid = lax.axis_index("s")

        @pl.when(jnp.logical_and(cid == 0, sid == 0))
        def _():
            pltpu.sync_copy(rank_hbm, rank_v)
            pltpu.sync_copy(hits_hbm, hits_v)

            def scatter_body(i, carry):
                kv = rank_v[pl.ds(i * 16, 16)]
                hv = hits_v[pl.ds(i * 16, 16)]
                plsc.store_scatter(hs_v, [kv], hv)
                return carry

            lax.fori_loop(0, N // 16, scatter_body, 0)

            def scan_body(i, carry):
                hv = hs_v[pl.ds(i * 16, 16)]
                cs = plsc.cumsum(hv) + carry
                cs_v[pl.ds(i * 16, 16)] = cs
                return carry + jnp.sum(hv)

            lax.fori_loop(0, N // 16, scan_body, jnp.float32(0.0))

            pltpu.sync_copy(hs_v, hs_out)
            pltpu.sync_copy(cs_v, cinc_out)

    return sc_kernel(rank, hits)


KT = 256


def _cummax_rows(T):
    s = 1
    while s < RB:
        T = jnp.maximum(T, jnp.concatenate(
            [jnp.full((s, KT), NEG_INF, jnp.float32), T[: RB - s, :]],
            axis=0))
        s *= 2
    return T


def _pava_body(cinc_ref, hs_ref, out_ref, hsb_ref, carry_ref, acc_ref):
    b = pl.program_id(0)
    base = b * RB

    @pl.when(b == 0)
    def _():
        carry_ref[...] = jnp.full((1, N), NEG_INF, jnp.float32)
        hsb_ref[...] = hs_ref[...] > 0.5

    sexc = (cinc_ref[0, pl.ds(base, RB)]
            - hs_ref[0, pl.ds(base, RB)]).reshape(RB, 1)
    dif = (lax.broadcasted_iota(jnp.int32, (RB, KT), 1)
           - lax.broadcasted_iota(jnp.int32, (RB, KT), 0)
           ).astype(jnp.float32)

    cinc_t = cinc_ref[0, pl.ds(base, KT)].reshape(1, KT)
    valid = dif >= 0.0
    M = jnp.where(valid, (cinc_t - sexc) / jnp.where(valid, dif + 1.0, 1.0),
                  NEG_INF)
    T = jnp.maximum(_cummax_rows(M), carry_ref[0, pl.ds(base, KT)]
                    .reshape(1, KT))
    carry_ref[0, pl.ds(base, KT)] = T[RB - 1, :]
    acc_ref[...] = jnp.min(jnp.where(valid, T, POS_INF), axis=1,
                           keepdims=True)

    def tile_body(t, c):
        k0 = t * KT
        cinc_t = cinc_ref[0, pl.ds(k0, KT)].reshape(1, KT)
        M = (cinc_t - sexc) / (dif + (k0 - base + 1.0))
        T = jnp.maximum(_cummax_rows(M), carry_ref[0, pl.ds(k0, KT)]
                        .reshape(1, KT))
        carry_ref[0, pl.ds(k0, KT)] = T[RB - 1, :]
        acc_ref[...] = jnp.minimum(acc_ref[...],
                                   jnp.min(T, axis=1, keepdims=True))
        return c

    lax.fori_loop(b + 1, N // KT, tile_body, 0)
    out_ref[...] = acc_ref[...]


def _pava(cinc, hs):
    return pl.pallas_call(
        _pava_body,
        grid=(NBLK,),
        in_specs=[
            pl.BlockSpec((1, N), lambda i: (0, 0)),
            pl.BlockSpec((1, N), lambda i: (0, 0)),
        ],
        out_specs=[pl.BlockSpec((RB, 1), lambda i: (i, 0)),
                   pl.BlockSpec((1, N), lambda i: (0, 0))],
        out_shape=[jax.ShapeDtypeStruct((N, 1), jnp.float32),
                   jax.ShapeDtypeStruct((1, N), jnp.bool_)],
        scratch_shapes=[pltpu.VMEM((1, N), jnp.float32),
                        pltpu.VMEM((RB, 1), jnp.float32)],
    )(cinc, hs)


HMAX = 128


def _sc_compact(conf, hits):
    mesh = plsc.VectorSubcoreMesh(core_axis_name="c", subcore_axis_name="s")

    @functools.partial(
        pl.kernel,
        mesh=mesh,
        compiler_params=pltpu.CompilerParams(needs_layout_passes=False),
        out_type=[
            jax.ShapeDtypeStruct((HMAX,), jnp.float32),
            jax.ShapeDtypeStruct((HMAX,), jnp.int32),
            jax.ShapeDtypeStruct((16,), jnp.float32),
        ],
        scratch_types=[
            pltpu.VMEM((N,), jnp.float32),
            pltpu.VMEM((N,), jnp.float32),
            pltpu.VMEM((HMAX,), jnp.float32),
            pltpu.VMEM((HMAX,), jnp.int32),
            pltpu.VMEM((16,), jnp.float32),
        ],
    )
    def sc_kernel(conf_hbm, hits_hbm, ch_out, ih_out, aux_out,
                  conf_v, hits_v, ch_v, ih_v, aux_v):
        cid = lax.axis_index("c")
        sid = lax.axis_index("s")

        @pl.when(jnp.logical_and(cid == 0, sid == 0))
        def _():
            pltpu.sync_copy(conf_hbm, conf_v)
            pltpu.sync_copy(hits_hbm, hits_v)

            def pad_body(i, c):
                ch_v[pl.ds(i * 16, 16)] = jnp.full((16,), 2.0, jnp.float32)
                ih_v[pl.ds(i * 16, 16)] = jnp.zeros((16,), jnp.int32)
                return c

            lax.fori_loop(0, HMAX // 16, pad_body, 0)

            lane = lax.iota(jnp.int32, 16)

            def comp_body(i, off):
                cv = conf_v[pl.ds(i * 16, 16)]
                hv = hits_v[pl.ds(i * 16, 16)]
                mask = hv > 0.5
                pos = off + plsc.cumsum(jnp.where(mask, 1, 0)) - 1
                plsc.store_scatter(ch_v, [pos], cv, mask=mask)
                plsc.store_scatter(ih_v, [pos], i * 16 + lane, mask=mask)
                npop = plsc.all_reduce_population_count(mask)
                return off + jnp.max(npop)

            cnt = lax.fori_loop(0, N // 16, comp_body, jnp.int32(0))
            aux_v[...] = jnp.where(lane == 0, cnt.astype(jnp.float32), 0.0)

            pltpu.sync_copy(ch_v, ch_out)
            pltpu.sync_copy(ih_v, ih_out)
            pltpu.sync_copy(aux_v, aux_out)

    return sc_kernel(conf, hits)


def _solve_body(chc_ref, ihc_ref, chr_ref, ihr_ref, aux_ref, conf_ref,
                out_ref, hsb_ref):
    cnt_i = aux_ref[0, 0].astype(jnp.int32)

    ch_c = chc_ref[...]
    ih_c = ihc_ref[...]
    ch_r = chr_ref[...]
    ih_r = ihr_ref[...]

    CHK = 1024
    jcol = lax.broadcasted_iota(jnp.int32, (HMAX, CHK), 1)

    def rank_chunk(t, racc):
        cj = conf_ref[0, pl.ds(t * CHK, CHK)].reshape(1, CHK)
        jj = t * CHK + jcol
        before = (cj < ch_c) | ((cj == ch_c) & (jj < ih_c))
        return racc + jnp.sum(before.astype(jnp.int32), axis=1, keepdims=True)

    rh_c = lax.fori_loop(0, N // CHK, rank_chunk,
                         jnp.zeros((HMAX, 1), jnp.int32))

    CHR = 512
    jrow2 = lax.broadcasted_iota(jnp.int32, (CHR, 1), 0)

    def rank_chunk_r(t, racc):
        cj = conf_ref[0, pl.ds(t * CHR, CHR)].reshape(CHR, 1)
        jj = t * CHR + jrow2
        before = (cj < ch_r) | ((cj == ch_r) & (jj < ih_r))
        return racc + jnp.sum(before.astype(jnp.int32), axis=0, keepdims=True)

    rh_r = lax.fori_loop(0, N // CHR, rank_chunk_r,
                         jnp.zeros((1, HMAX), jnp.int32))

    hcol = lax.broadcasted_iota(jnp.int32, (HMAX, 1), 0)
    hrow = lax.broadcasted_iota(jnp.int32, (1, HMAX), 1)
    before2 = (rh_r < rh_c) | ((rh_r == rh_c) & (hrow < hcol))
    rr_c = jnp.sum(before2.astype(jnp.int32), axis=1, keepdims=True)
    before2t = (rh_c < rh_r) | ((rh_c == rh_r) & (hcol < hrow))
    rr_r = jnp.sum(before2t.astype(jnp.int32), axis=0, keepdims=True)

    m_hits_r = jnp.sum(rh_c.astype(jnp.float32)
                       * (rr_c == hrow).astype(jnp.float32),
                       axis=0, keepdims=True)
    m_hits_c = jnp.sum(rh_r.astype(jnp.float32)
                       * (rr_r == hcol).astype(jnp.float32),
                       axis=1, keepdims=True)

    candm_r = jnp.concatenate(
        [jnp.zeros((1, 1), jnp.float32), m_hits_r[:, : HMAX - 1]], axis=1)
    candm_c = jnp.concatenate(
        [jnp.zeros((1, 1), jnp.float32), m_hits_c[: HMAX - 1, :]], axis=0)
    candS_r = jnp.maximum(hrow - 1, 0).astype(jnp.float32)
    candS_c = jnp.maximum(hcol - 1, 0).astype(jnp.float32)

    fh = jnp.sum(jnp.where(hrow == 0, m_hits_r, 0.0))
    r0_ok = fh != 0.0
    vr_c = (hcol <= cnt_i) & ((hcol >= 1) | r0_ok)
    validq = (hrow >= 1) & (hrow <= cnt_i + 1)

    qgtr = hrow > hcol
    M = jnp.where(qgtr & validq & vr_c,
                  (candS_r - candS_c) / (candm_r - candm_c), NEG_INF)

    s = 1
    while s < HMAX:
        M = jnp.maximum(M, jnp.concatenate(
            [jnp.full((s, HMAX), NEG_INF, jnp.float32), M[: HMAX - s, :]],
            axis=0))
        s *= 2
    slope_c = jnp.min(jnp.where(qgtr & validq, M, POS_INF), axis=1,
                      keepdims=True)

    lanes = lax.broadcasted_iota(jnp.int32, (1, RB), 1)
    cntmask_c = hcol <= cnt_i + 1

    def map_body(b2, c):
        i_row = (b2 * RB + lanes).astype(jnp.float32)
        pi = jnp.sum(((candm_c <= i_row) & cntmask_c).astype(jnp.int32),
                     axis=0, keepdims=True) - 1
        iso = jnp.sum(jnp.where(hcol == pi, slope_c, 0.0), axis=0,
                      keepdims=True)
        out_ref[0, pl.ds(b2 * RB, RB)] = iso[0, :]
        kk = b2 * RB + lanes
        hb = jnp.sum((rh_c == kk).astype(jnp.int32), axis=0,
                     keepdims=True) > 0
        hsb_ref[0, pl.ds(b2 * RB, RB)] = hb[0, :]
        return c

    lax.fori_loop(0, N // RB, map_body, 0)


def _solve(ch, ih, aux, conf):
    return pl.pallas_call(
        _solve_body,
        grid=(1,),
        in_specs=[
            pl.BlockSpec((HMAX, 1), lambda i: (0, 0)),
            pl.BlockSpec((HMAX, 1), lambda i: (0, 0)),
            pl.BlockSpec((1, HMAX), lambda i: (0, 0)),
            pl.BlockSpec((1, HMAX), lambda i: (0, 0)),
            pl.BlockSpec(memory_space=pltpu.SMEM),
            pl.BlockSpec((1, N), lambda i: (0, 0)),
        ],
        out_specs=[pl.BlockSpec((1, N), lambda i: (0, 0)),
                   pl.BlockSpec((1, N), lambda i: (0, 0))],
        out_shape=[jax.ShapeDtypeStruct((1, N), jnp.float32),
                   jax.ShapeDtypeStruct((1, N), jnp.bool_)],
    )(ch.reshape(HMAX, 1), ih.reshape(HMAX, 1),
      ch.reshape(1, HMAX), ih.reshape(1, HMAX), aux, conf)


def kernel(Simple_vector, label_list):
    lab2 = label_list.reshape(1, N)
    conf, hits = _stats(Simple_vector, lab2)

    def sparse_path(_):
        ch, ih, aux = _sc_compact(conf.reshape(N), hits.reshape(N))
        cali, hb = _solve(ch, ih, aux.reshape(1, 16), conf)
        return cali.reshape(N), hb.reshape(N)

    def dense_path(_):
        rank = _ranks(conf)
        hits_s, cinc = _sc_scatter_scan(rank.reshape(N), hits.reshape(N))
        cali, hb = _pava(cinc.reshape(1, N), hits_s.reshape(1, N))
        return cali.reshape(N), hb.reshape(N)

    nhits = jnp.sum(hits)
    return lax.cond(nhits <= HMAX - 2.0, sparse_path, dense_path, 0)


__all__ = ["kernel"]


# speedup vs baseline: 2.1208x; 1.0506x over previous
import functools

import jax
import jax.numpy as jnp
from jax import lax
from jax.experimental import pallas as pl
from jax.experimental.pallas import tpu as pltpu
from jax.experimental.pallas import tpu_sc as plsc

N = 4096
C = 1000
RB = 256
NBLK = N // RB
NEG_INF = float("-inf")
POS_INF = float("inf")


def _stats_body(x_ref, lab_ref, conf_ref, hits_ref):
    x = x_ref[...]
    lab = lab_ref[0, :]
    maxv = jnp.max(x, axis=1, keepdims=True)
    s = jnp.sum(jnp.exp(x - maxv), axis=1)
    col = lax.broadcasted_iota(jnp.int32, x.shape, 1)
    am = jnp.min(jnp.where(x >= maxv, col, C), axis=1)
    conf_ref[0, :] = 1.0 / s
    hits_ref[0, :] = (am == lab).astype(jnp.float32)


SB = 1024


def _stats(x, lab2):
    return pl.pallas_call(
        _stats_body,
        grid=(N // SB,),
        in_specs=[
            pl.BlockSpec((SB, C), lambda i: (i, 0)),
            pl.BlockSpec((1, SB), lambda i: (0, i)),
        ],
        out_specs=[
            pl.BlockSpec((1, SB), lambda i: (0, i)),
            pl.BlockSpec((1, SB), lambda i: (0, i)),
        ],
        out_shape=[
            jax.ShapeDtypeStruct((1, N), jnp.float32),
            jax.ShapeDtypeStruct((1, N), jnp.float32),
        ],
    )(x, lab2)


def _rank_body(conf_ref, rank_ref):
    i = pl.program_id(0)
    cj = conf_ref[0, :].reshape(1, N)
    ci = conf_ref[0, pl.ds(i * RB, RB)].reshape(RB, 1)
    jidx = lax.broadcasted_iota(jnp.int32, (RB, N), 1)
    iidx = i * RB + lax.broadcasted_iota(jnp.int32, (RB, N), 0)
    before = (cj < ci) | ((cj == ci) & (jidx < iidx))
    rank_ref[0, :] = jnp.sum(before.astype(jnp.int32), axis=1)


def _ranks(conf):
    return pl.pallas_call(
        _rank_body,
        grid=(NBLK,),
        in_specs=[pl.BlockSpec((1, N), lambda i: (0, 0))],
        out_specs=pl.BlockSpec((1, RB), lambda i: (0, i)),
        out_shape=jax.ShapeDtypeStruct((1, N), jnp.int32),
    )(conf)


def _sc_scatter_scan(rank, hits):
    mesh = plsc.VectorSubcoreMesh(core_axis_name="c", subcore_axis_name="s")

    @functools.partial(
        pl.kernel,
        mesh=mesh,
        compiler_params=pltpu.CompilerParams(needs_layout_passes=False),
        out_type=[
            jax.ShapeDtypeStruct((N,), jnp.float32),
            jax.ShapeDtypeStruct((N,), jnp.float32),
        ],
        scratch_types=[
            pltpu.VMEM((N,), jnp.int32),
            pltpu.VMEM((N,), jnp.float32),
            pltpu.VMEM((N,), jnp.float32),
            pltpu.VMEM((N,), jnp.float32),
        ],
    )
    def sc_kernel(rank_hbm, hits_hbm, hs_out, cinc_out, rank_v, hits_v,
                  hs_v, cs_v):
        cid = lax.axis_index("c")
        sid = lax.axis_index("s")

        @pl.when(jnp.logical_and(cid == 0, sid == 0))
        def _():
            pltpu.sync_copy(rank_hbm, rank_v)
            pltpu.sync_copy(hits_hbm, hits_v)

            def scatter_body(i, carry):
                kv = rank_v[pl.ds(i * 16, 16)]
                hv = hits_v[pl.ds(i * 16, 16)]
                plsc.store_scatter(hs_v, [kv], hv)
                return carry

            lax.fori_loop(0, N // 16, scatter_body, 0)

            def scan_body(i, carry):
                hv = hs_v[pl.ds(i * 16, 16)]
                cs = plsc.cumsum(hv) + carry
                cs_v[pl.ds(i * 16, 16)] = cs
                return carry + jnp.sum(hv)

            lax.fori_loop(0, N // 16, scan_body, jnp.float32(0.0))

            pltpu.sync_copy(hs_v, hs_out)
            pltpu.sync_copy(cs_v, cinc_out)

    return sc_kernel(rank, hits)


KT = 256


def _cummax_rows(T):
    s = 1
    while s < RB:
        T = jnp.maximum(T, jnp.concatenate(
            [jnp.full((s, KT), NEG_INF, jnp.float32), T[: RB - s, :]],
            axis=0))
        s *= 2
    return T


def _pava_body(cinc_ref, hs_ref, out_ref, hsb_ref, carry_ref, acc_ref):
    b = pl.program_id(0)
    base = b * RB

    @pl.when(b == 0)
    def _():
        carry_ref[...] = jnp.full((1, N), NEG_INF, jnp.float32)
        hsb_ref[...] = hs_ref[...] > 0.5

    sexc = (cinc_ref[0, pl.ds(base, RB)]
            - hs_ref[0, pl.ds(base, RB)]).reshape(RB, 1)
    dif = (lax.broadcasted_iota(jnp.int32, (RB, KT), 1)
           - lax.broadcasted_iota(jnp.int32, (RB, KT), 0)
           ).astype(jnp.float32)

    cinc_t = cinc_ref[0, pl.ds(base, KT)].reshape(1, KT)
    valid = dif >= 0.0
    M = jnp.where(valid, (cinc_t - sexc) / jnp.where(valid, dif + 1.0, 1.0),
                  NEG_INF)
    T = jnp.maximum(_cummax_rows(M), carry_ref[0, pl.ds(base, KT)]
                    .reshape(1, KT))
    carry_ref[0, pl.ds(base, KT)] = T[RB - 1, :]
    acc_ref[...] = jnp.min(jnp.where(valid, T, POS_INF), axis=1,
                           keepdims=True)

    def tile_body(t, c):
        k0 = t * KT
        cinc_t = cinc_ref[0, pl.ds(k0, KT)].reshape(1, KT)
        M = (cinc_t - sexc) / (dif + (k0 - base + 1.0))
        T = jnp.maximum(_cummax_rows(M), carry_ref[0, pl.ds(k0, KT)]
                        .reshape(1, KT))
        carry_ref[0, pl.ds(k0, KT)] = T[RB - 1, :]
        acc_ref[...] = jnp.minimum(acc_ref[...],
                                   jnp.min(T, axis=1, keepdims=True))
        return c

    lax.fori_loop(b + 1, N // KT, tile_body, 0)
    out_ref[...] = acc_ref[...]


def _pava(cinc, hs):
    return pl.pallas_call(
        _pava_body,
        grid=(NBLK,),
        in_specs=[
            pl.BlockSpec((1, N), lambda i: (0, 0)),
            pl.BlockSpec((1, N), lambda i: (0, 0)),
        ],
        out_specs=[pl.BlockSpec((RB, 1), lambda i: (i, 0)),
                   pl.BlockSpec((1, N), lambda i: (0, 0))],
        out_shape=[jax.ShapeDtypeStruct((N, 1), jnp.float32),
                   jax.ShapeDtypeStruct((1, N), jnp.bool_)],
        scratch_shapes=[pltpu.VMEM((1, N), jnp.float32),
                        pltpu.VMEM((RB, 1), jnp.float32)],
    )(cinc, hs)


HMAX = 128


def _sc_compact(conf, hits):
    mesh = plsc.VectorSubcoreMesh(core_axis_name="c", subcore_axis_name="s")

    @functools.partial(
        pl.kernel,
        mesh=mesh,
        compiler_params=pltpu.CompilerParams(needs_layout_passes=False),
        out_type=[
            jax.ShapeDtypeStruct((HMAX,), jnp.float32),
            jax.ShapeDtypeStruct((HMAX,), jnp.int32),
            jax.ShapeDtypeStruct((16,), jnp.float32),
        ],
        scratch_types=[
            pltpu.VMEM((N,), jnp.float32),
            pltpu.VMEM((N,), jnp.float32),
            pltpu.VMEM((HMAX,), jnp.float32),
            pltpu.VMEM((HMAX,), jnp.int32),
            pltpu.VMEM((16,), jnp.float32),
        ],
    )
    def sc_kernel(conf_hbm, hits_hbm, ch_out, ih_out, aux_out,
                  conf_v, hits_v, ch_v, ih_v, aux_v):
        cid = lax.axis_index("c")
        sid = lax.axis_index("s")

        @pl.when(jnp.logical_and(cid == 0, sid == 0))
        def _():
            pltpu.sync_copy(conf_hbm, conf_v)
            pltpu.sync_copy(hits_hbm, hits_v)

            def pad_body(i, c):
                ch_v[pl.ds(i * 16, 16)] = jnp.full((16,), 2.0, jnp.float32)
                ih_v[pl.ds(i * 16, 16)] = jnp.zeros((16,), jnp.int32)
                return c

            lax.fori_loop(0, HMAX // 16, pad_body, 0)

            lane = lax.iota(jnp.int32, 16)

            def comp_body(i, off):
                cv = conf_v[pl.ds(i * 16, 16)]
                hv = hits_v[pl.ds(i * 16, 16)]
                mask = hv > 0.5
                pos = off + plsc.cumsum(jnp.where(mask, 1, 0)) - 1
                wmask = jnp.logical_and(mask, pos < HMAX)
                plsc.store_scatter(ch_v, [pos], cv, mask=wmask)
                plsc.store_scatter(ih_v, [pos], i * 16 + lane, mask=wmask)
                npop = plsc.all_reduce_population_count(mask)
                return off + jnp.max(npop)

            cnt = lax.fori_loop(0, N // 16, comp_body, jnp.int32(0))
            aux_v[...] = jnp.where(lane == 0, cnt.astype(jnp.float32), 0.0)

            pltpu.sync_copy(ch_v, ch_out)
            pltpu.sync_copy(ih_v, ih_out)
            pltpu.sync_copy(aux_v, aux_out)

    return sc_kernel(conf, hits)


def _solve_body(chc_ref, ihc_ref, chr_ref, ihr_ref, aux_ref, conf_ref,
                out_ref, hsb_ref):
    cnt_i = aux_ref[0, 0].astype(jnp.int32)

    ch_c = chc_ref[...]
    ih_c = ihc_ref[...]
    ch_r = chr_ref[...]
    ih_r = ihr_ref[...]

    CHK = 1024
    jcol = lax.broadcasted_iota(jnp.int32, (HMAX, CHK), 1)

    def rank_chunk(t, racc):
        cj = conf_ref[0, pl.ds(t * CHK, CHK)].reshape(1, CHK)
        jj = t * CHK + jcol
        before = (cj < ch_c) | ((cj == ch_c) & (jj < ih_c))
        return racc + jnp.sum(before.astype(jnp.int32), axis=1, keepdims=True)

    rh_c = lax.fori_loop(0, N // CHK, rank_chunk,
                         jnp.zeros((HMAX, 1), jnp.int32))

    CHR = 512
    jrow2 = lax.broadcasted_iota(jnp.int32, (CHR, 1), 0)

    def rank_chunk_r(t, racc):
        cj = conf_ref[0, pl.ds(t * CHR, CHR)].reshape(CHR, 1)
        jj = t * CHR + jrow2
        before = (cj < ch_r) | ((cj == ch_r) & (jj < ih_r))
        return racc + jnp.sum(before.astype(jnp.int32), axis=0, keepdims=True)

    rh_r = lax.fori_loop(0, N // CHR, rank_chunk_r,
                         jnp.zeros((1, HMAX), jnp.int32))

    hcol = lax.broadcasted_iota(jnp.int32, (HMAX, 1), 0)
    hrow = lax.broadcasted_iota(jnp.int32, (1, HMAX), 1)
    before2 = (rh_r < rh_c) | ((rh_r == rh_c) & (hrow < hcol))
    rr_c = jnp.sum(before2.astype(jnp.int32), axis=1, keepdims=True)
    before2t = (rh_c < rh_r) | ((rh_c == rh_r) & (hcol < hrow))
    rr_r = jnp.sum(before2t.astype(jnp.int32), axis=0, keepdims=True)

    m_hits_r = jnp.sum(rh_c.astype(jnp.float32)
                       * (rr_c == hrow).astype(jnp.float32),
                       axis=0, keepdims=True)
    m_hits_c = jnp.sum(rh_r.astype(jnp.float32)
                       * (rr_r == hcol).astype(jnp.float32),
                       axis=1, keepdims=True)

    candm_r = jnp.concatenate(
        [jnp.zeros((1, 1), jnp.float32), m_hits_r[:, : HMAX - 1]], axis=1)
    candm_c = jnp.concatenate(
        [jnp.zeros((1, 1), jnp.float32), m_hits_c[: HMAX - 1, :]], axis=0)
    candS_r = jnp.maximum(hrow - 1, 0).astype(jnp.float32)
    candS_c = jnp.maximum(hcol - 1, 0).astype(jnp.float32)

    fh = jnp.sum(jnp.where(hrow == 0, m_hits_r, 0.0))
    r0_ok = fh != 0.0
    vr_c = (hcol <= cnt_i) & ((hcol >= 1) | r0_ok)
    validq = (hrow >= 1) & (hrow <= cnt_i + 1)

    qgtr = hrow > hcol
    M = jnp.where(qgtr & validq & vr_c,
                  (candS_r - candS_c) / (candm_r - candm_c), NEG_INF)

    s = 1
    while s < HMAX:
        M = jnp.maximum(M, jnp.concatenate(
            [jnp.full((s, HMAX), NEG_INF, jnp.float32), M[: HMAX - s, :]],
            axis=0))
        s *= 2
    slope_c = jnp.min(jnp.where(qgtr & validq, M, POS_INF), axis=1,
                      keepdims=True)

    lanes = lax.broadcasted_iota(jnp.int32, (1, RB), 1)
    cntmask_c = hcol <= cnt_i + 1

    def map_body(b2, c):
        i_row = (b2 * RB + lanes).astype(jnp.float32)
        pi = jnp.sum(((candm_c <= i_row) & cntmask_c).astype(jnp.int32),
                     axis=0, keepdims=True) - 1
        iso = jnp.sum(jnp.where(hcol == pi, slope_c, 0.0), axis=0,
                      keepdims=True)
        out_ref[0, pl.ds(b2 * RB, RB)] = iso[0, :]
        kk = b2 * RB + lanes
        hb = jnp.sum((rh_c == kk).astype(jnp.int32), axis=0,
                     keepdims=True) > 0
        hsb_ref[0, pl.ds(b2 * RB, RB)] = hb[0, :]
        return c

    lax.fori_loop(0, N // RB, map_body, 0)


def _solve(ch, ih, aux, conf):
    return pl.pallas_call(
        _solve_body,
        grid=(1,),
        in_specs=[
            pl.BlockSpec((HMAX, 1), lambda i: (0, 0)),
            pl.BlockSpec((HMAX, 1), lambda i: (0, 0)),
            pl.BlockSpec((1, HMAX), lambda i: (0, 0)),
            pl.BlockSpec((1, HMAX), lambda i: (0, 0)),
            pl.BlockSpec(memory_space=pltpu.SMEM),
            pl.BlockSpec((1, N), lambda i: (0, 0)),
        ],
        out_specs=[pl.BlockSpec((1, N), lambda i: (0, 0)),
                   pl.BlockSpec((1, N), lambda i: (0, 0))],
        out_shape=[jax.ShapeDtypeStruct((1, N), jnp.float32),
                   jax.ShapeDtypeStruct((1, N), jnp.bool_)],
    )(ch.reshape(HMAX, 1), ih.reshape(HMAX, 1),
      ch.reshape(1, HMAX), ih.reshape(1, HMAX), aux, conf)


def kernel(Simple_vector, label_list):
    lab2 = label_list.reshape(1, N)
    conf, hits = _stats(Simple_vector, lab2)

    ch, ih, aux = _sc_compact(conf.reshape(N), hits.reshape(N))

    def sparse_path(_):
        cali, hb = _solve(ch, ih, aux.reshape(1, 16), conf)
        return cali.reshape(N), hb.reshape(N)

    def dense_path(_):
        rank = _ranks(conf)
        hits_s, cinc = _sc_scatter_scan(rank.reshape(N), hits.reshape(N))
        cali, hb = _pava(cinc.reshape(1, N), hits_s.reshape(1, N))
        return cali.reshape(N), hb.reshape(N)

    return lax.cond(aux[0] <= HMAX - 2.0, sparse_path, dense_path, 0)


__all__ = ["kernel"]
